# baseline (speedup 1.0000x reference)
import jax
import jax.numpy as jnp
from jax.experimental import pallas as pl
from jax.experimental.pallas import tpu as pltpu

_NORM = 1.0


def _decoder_kernel(a_ref, h_ref, mask_ref,
                    wm0_ref, bm0_ref, wu0_ref, bu0_ref,
                    wm1_ref, bm1_ref, wu1_ref, bu1_ref,
                    wo_ref, bo_ref, out_ref, h_scr):
    f32 = jnp.float32
    P = jax.lax.Precision.DEFAULT
    layer = jax.lax.rem(pl.program_id(0), 2)
    is_l0 = layer == 0

    h = jnp.where(is_l0, h_ref[0], h_scr[...])
    wm = jnp.where(is_l0, wm0_ref[...], wm1_ref[...])
    bm = jnp.where(is_l0, bm0_ref[...], bm1_ref[...])
    wu = jnp.where(is_l0, wu0_ref[...], wu1_ref[...])
    bu = jnp.where(is_l0, bu0_ref[...], bu1_ref[...])

    m = jnp.dot(h, wm, precision=P, preferred_element_type=f32) + bm
    agg = jnp.dot(a_ref[0], m, precision=P, preferred_element_type=f32)
    agg = agg * (1.0 / _NORM)
    h_next = jnp.maximum(
        jnp.dot(agg, wu, precision=P, preferred_element_type=f32) + bu, 0.0)
    h_scr[...] = h_next

    @pl.when(layer == 1)
    def _():
        out = jnp.dot(h_next, wo_ref[...], precision=P,
                      preferred_element_type=f32) + bo_ref[...]
        out_ref[0] = out * mask_ref[0]


def kernel(latent_features, adjacency_matrix, node_mask,
           W_msg0, b_msg0, W_upd0, b_upd0,
           W_msg1, b_msg1, W_upd1, b_upd1,
           W_out, b_out):
    B, N, d_lat = latent_features.shape
    d_hid = W_msg0.shape[1]
    d_out = W_out.shape[1]

    b2 = lambda b: b.reshape(1, -1)

    batch_spec = lambda shape: pl.BlockSpec(shape, lambda i: (i // 2, 0, 0))
    w_spec = pl.BlockSpec((d_hid, d_hid), lambda i: (0, 0))
    bias_spec = pl.BlockSpec((1, d_hid), lambda i: (0, 0))

    return pl.pallas_call(
        _decoder_kernel,
        grid=(2 * B,),
        in_specs=[
            batch_spec((1, N, N)),
            batch_spec((1, N, d_lat)),
            batch_spec((1, N, 1)),
            pl.BlockSpec((d_lat, d_hid), lambda i: (0, 0)), bias_spec,
            w_spec, bias_spec,
            w_spec, bias_spec,
            w_spec, bias_spec,
            pl.BlockSpec((d_hid, d_out), lambda i: (0, 0)),
            pl.BlockSpec((1, d_out), lambda i: (0, 0)),
        ],
        out_specs=batch_spec((1, N, d_out)),
        out_shape=jax.ShapeDtypeStruct((B, N, d_out), jnp.float32),
        scratch_shapes=[pltpu.VMEM((N, d_hid), jnp.float32)],
        compiler_params=pltpu.CompilerParams(
            dimension_semantics=("arbitrary",),
            vmem_limit_bytes=64 * 1024 * 1024,
        ),
    )(adjacency_matrix, latent_features, node_mask,
      W_msg0, b2(b_msg0), W_upd0, b2(b_upd0),
      W_msg1, b2(b_msg1), W_upd1, b2(b_upd1),
      W_out, b2(b_out))


# speedup vs baseline: 1.1833x; 1.1833x over previous
import jax
import jax.numpy as jnp
from jax.experimental import pallas as pl
from jax.experimental.pallas import tpu as pltpu

_NORM = 1.0


def _decoder_kernel(a_hbm, h_ref, mask_ref,
                    wm0_ref, bm0_ref, wu0_ref, bu0_ref,
                    wm1_ref, bm1_ref, wu1_ref, bu1_ref,
                    wo_ref, bo_ref, out_ref, h_scr, a_vmem, sems):
    f32 = jnp.float32
    P = jax.lax.Precision.DEFAULT
    i = pl.program_id(0)
    num_b = pl.num_programs(0) // 2
    b = i // 2
    layer = jax.lax.rem(i, 2)
    slot = jax.lax.rem(b, 2)
    nslot = jax.lax.rem(b + 1, 2)

    @pl.when(i == 0)
    def _():
        pltpu.make_async_copy(a_hbm.at[0], a_vmem.at[0], sems.at[0]).start()

    @pl.when((layer == 0) & (b + 1 < num_b))
    def _():
        pltpu.make_async_copy(a_hbm.at[b + 1], a_vmem.at[nslot],
                              sems.at[nslot]).start()

    @pl.when(layer == 0)
    def _():
        pltpu.make_async_copy(a_hbm.at[b], a_vmem.at[slot],
                              sems.at[slot]).wait()

    is_l0 = layer == 0
    h = jnp.where(is_l0, h_ref[0], h_scr[...])
    wm = jnp.where(is_l0, wm0_ref[...], wm1_ref[...])
    bm = jnp.where(is_l0, bm0_ref[...], bm1_ref[...])
    wu = jnp.where(is_l0, wu0_ref[...], wu1_ref[...])
    bu = jnp.where(is_l0, bu0_ref[...], bu1_ref[...])

    m = jnp.dot(h, wm, precision=P, preferred_element_type=f32) + bm
    agg = jnp.dot(a_vmem[slot], m, precision=P, preferred_element_type=f32)
    agg = agg * (1.0 / _NORM)
    h_next = jnp.maximum(
        jnp.dot(agg, wu, precision=P, preferred_element_type=f32) + bu, 0.0)
    h_scr[...] = h_next

    @pl.when(layer == 1)
    def _():
        out = jnp.dot(h_next, wo_ref[...], precision=P,
                      preferred_element_type=f32) + bo_ref[...]
        out_ref[0] = out * mask_ref[0]


def kernel(latent_features, adjacency_matrix, node_mask,
           W_msg0, b_msg0, W_upd0, b_upd0,
           W_msg1, b_msg1, W_upd1, b_upd1,
           W_out, b_out):
    B, N, d_lat = latent_features.shape
    d_hid = W_msg0.shape[1]
    d_out = W_out.shape[1]

    b2 = lambda b: b.reshape(1, -1)

    batch_spec = lambda shape: pl.BlockSpec(shape, lambda i: (i // 2, 0, 0))
    w_spec = pl.BlockSpec((d_hid, d_hid), lambda i: (0, 0))
    bias_spec = pl.BlockSpec((1, d_hid), lambda i: (0, 0))

    return pl.pallas_call(
        _decoder_kernel,
        grid=(2 * B,),
        in_specs=[
            pl.BlockSpec(memory_space=pltpu.MemorySpace.HBM),
            batch_spec((1, N, d_lat)),
            batch_spec((1, N, 1)),
            pl.BlockSpec((d_lat, d_hid), lambda i: (0, 0)), bias_spec,
            w_spec, bias_spec,
            w_spec, bias_spec,
            w_spec, bias_spec,
            pl.BlockSpec((d_hid, d_out), lambda i: (0, 0)),
            pl.BlockSpec((1, d_out), lambda i: (0, 0)),
        ],
        out_specs=batch_spec((1, N, d_out)),
        out_shape=jax.ShapeDtypeStruct((B, N, d_out), jnp.float32),
        scratch_shapes=[
            pltpu.VMEM((N, d_hid), jnp.float32),
            pltpu.VMEM((2, N, N), jnp.float32),
            pltpu.SemaphoreType.DMA((2,)),
        ],
        compiler_params=pltpu.CompilerParams(
            dimension_semantics=("arbitrary",),
            vmem_limit_bytes=64 * 1024 * 1024,
        ),
    )(adjacency_matrix, latent_features, node_mask,
      W_msg0, b2(b_msg0), W_upd0, b2(b_upd0),
      W_msg1, b2(b_msg1), W_upd1, b2(b_upd1),
      W_out, b2(b_out))


# speedup vs baseline: 1.1873x; 1.0034x over previous
import jax
import jax.numpy as jnp
from jax.experimental import pallas as pl
from jax.experimental.pallas import tpu as pltpu

_NORM = 1.0


def _decoder_kernel(a_hbm, h_ref, mask_ref,
                    wm0_ref, bm0_ref, wu0_ref, bu0_ref,
                    wm1_ref, bm1_ref, wu1_ref, bu1_ref,
                    wo_ref, bo_ref, out_ref, m_scr, a_vmem, sems):
    f32 = jnp.float32
    P = jax.lax.Precision.DEFAULT
    i = pl.program_id(0)
    num_b = pl.num_programs(0) // 2
    b = i // 2
    layer = jax.lax.rem(i, 2)
    slot = jax.lax.rem(b, 2)
    nslot = jax.lax.rem(b + 1, 2)

    @pl.when(i == 0)
    def _():
        pltpu.make_async_copy(a_hbm.at[0], a_vmem.at[0], sems.at[0]).start()

    @pl.when((layer == 0) & (b + 1 < num_b))
    def _():
        pltpu.make_async_copy(a_hbm.at[b + 1], a_vmem.at[nslot],
                              sems.at[nslot]).start()

    @pl.when(i == 0)
    def _():
        m_scr[...] = jnp.dot(h_ref[0], wm0_ref[...], precision=P,
                             preferred_element_type=f32) + bm0_ref[...]

    @pl.when(layer == 0)
    def _():
        pltpu.make_async_copy(a_hbm.at[b], a_vmem.at[slot],
                              sems.at[slot]).wait()

    is_l0 = layer == 0
    wu = jnp.where(is_l0, wu0_ref[...], wu1_ref[...])
    bu = jnp.where(is_l0, bu0_ref[...], bu1_ref[...])

    agg = jnp.dot(a_vmem[slot], m_scr[...], precision=P,
                  preferred_element_type=f32)
    agg = agg * (1.0 / _NORM)
    h_next = jnp.maximum(
        jnp.dot(agg, wu, precision=P, preferred_element_type=f32) + bu, 0.0)

    @pl.when(layer == 0)
    def _():
        m_scr[...] = jnp.dot(h_next, wm1_ref[...], precision=P,
                             preferred_element_type=f32) + bm1_ref[...]

    @pl.when((layer == 1) & (i + 1 < pl.num_programs(0)))
    def _():
        m_scr[...] = jnp.dot(h_ref[0], wm0_ref[...], precision=P,
                             preferred_element_type=f32) + bm0_ref[...]

    @pl.when(layer == 1)
    def _():
        out = jnp.dot(h_next, wo_ref[...], precision=P,
                      preferred_element_type=f32) + bo_ref[...]
        out_ref[0] = out * mask_ref[0]


def kernel(latent_features, adjacency_matrix, node_mask,
           W_msg0, b_msg0, W_upd0, b_upd0,
           W_msg1, b_msg1, W_upd1, b_upd1,
           W_out, b_out):
    B, N, d_lat = latent_features.shape
    d_hid = W_msg0.shape[1]
    d_out = W_out.shape[1]

    b2 = lambda b: b.reshape(1, -1)

    batch_spec = lambda shape: pl.BlockSpec(shape, lambda i: (i // 2, 0, 0))
    lat_spec = pl.BlockSpec((1, N, d_lat),
                            lambda i: (jnp.minimum((i + 1) // 2, B - 1), 0, 0))
    w_spec = pl.BlockSpec((d_hid, d_hid), lambda i: (0, 0))
    bias_spec = pl.BlockSpec((1, d_hid), lambda i: (0, 0))

    return pl.pallas_call(
        _decoder_kernel,
        grid=(2 * B,),
        in_specs=[
            pl.BlockSpec(memory_space=pltpu.MemorySpace.HBM),
            lat_spec,
            batch_spec((1, N, 1)),
            pl.BlockSpec((d_lat, d_hid), lambda i: (0, 0)), bias_spec,
            w_spec, bias_spec,
            w_spec, bias_spec,
            w_spec, bias_spec,
            pl.BlockSpec((d_hid, d_out), lambda i: (0, 0)),
            pl.BlockSpec((1, d_out), lambda i: (0, 0)),
        ],
        out_specs=batch_spec((1, N, d_out)),
        out_shape=jax.ShapeDtypeStruct((B, N, d_out), jnp.float32),
        scratch_shapes=[
            pltpu.VMEM((N, d_hid), jnp.float32),
            pltpu.VMEM((2, N, N), jnp.float32),
            pltpu.SemaphoreType.DMA((2,)),
        ],
        compiler_params=pltpu.CompilerParams(
            dimension_semantics=("arbitrary",),
            vmem_limit_bytes=64 * 1024 * 1024,
        ),
    )(adjacency_matrix, latent_features, node_mask,
      W_msg0, b2(b_msg0), W_upd0, b2(b_upd0),
      W_msg1, b2(b_msg1), W_upd1, b2(b_upd1),
      W_out, b2(b_out))
